# baseline (speedup 1.0000x reference)
import jax
import jax.numpy as jnp
from jax.experimental import pallas as pl
from jax.experimental.pallas import tpu as pltpu

B, S, K, D = 1024, 50, 20, 128
K2 = 24
BB = 128
R = BB * K2


def _entity_kernel(x_ref, m_ref, keys_ref, P_ref, UWV_ref, out_ref, C_ref):
    P = P_ref[...]
    UWV = UWV_ref[...]

    C_ref[:, 0:D] = jnp.zeros((R, D), dtype=jnp.float32)
    C_ref[:, 2 * D:3 * D] = keys_ref[...].reshape(R, D)

    def step(t, _):
        x_t = x_ref[t]
        m_t = m_ref[t].reshape(BB, 1)
        C_ref[:, D:2 * D] = jnp.dot(P, x_t,
                                    preferred_element_type=jnp.float32)
        h2 = C_ref[:, 0:D]
        xb = C_ref[:, D:2 * D]
        k2 = C_ref[:, 2 * D:3 * D]
        T = jnp.dot(C_ref[...], UWV,
                    preferred_element_type=jnp.float32)
        h_tilda = jax.nn.relu(T).reshape(BB, K2, D)
        z = jnp.sum((xb * (h2 + k2)).reshape(BB, K2, D), axis=2)
        g = m_t * (0.5 * jnp.tanh(0.5 * z) + 0.5)
        upd = h2.reshape(BB, K2, D) + g[..., None] * h_tilda
        inv = jax.lax.rsqrt(jnp.maximum(
            jnp.sum(upd * upd, axis=2, keepdims=True), 1e-12))
        C_ref[:, 0:D] = (upd * inv).reshape(R, D)
        return 0

    jax.lax.fori_loop(0, S, step, 0)
    out_ref[...] = C_ref[:, 0:D].reshape(BB, K2, D)[:, :K, :]


@jax.jit
def kernel(encoded_sents, mask, keys, U, V, W):
    x_t_first = jnp.swapaxes(encoded_sents, 0, 1)
    mask_f = jnp.swapaxes(mask, 0, 1).astype(jnp.float32)[:, None, :]
    keys_p = jnp.pad(keys, ((0, 0), (0, K2 - K), (0, 0)))
    P = jnp.repeat(jnp.eye(BB, dtype=jnp.float32), K2, axis=0)
    UWV = jnp.concatenate([U, W, V], axis=0)
    grid = (B // BB,)
    return pl.pallas_call(
        _entity_kernel,
        grid=grid,
        in_specs=[
            pl.BlockSpec((S, BB, D), lambda i: (0, i, 0)),
            pl.BlockSpec((S, 1, BB), lambda i: (0, 0, i)),
            pl.BlockSpec((BB, K2, D), lambda i: (i, 0, 0)),
            pl.BlockSpec((R, BB), lambda i: (0, 0)),
            pl.BlockSpec((3 * D, D), lambda i: (0, 0)),
        ],
        out_specs=pl.BlockSpec((BB, K, D), lambda i: (i, 0, 0)),
        out_shape=jax.ShapeDtypeStruct((B, K, D), jnp.float32),
        scratch_shapes=[pltpu.VMEM((R, 3 * D), jnp.float32)],
    )(x_t_first, mask_f, keys_p, P, UWV)
